# baseline (speedup 1.0000x reference)
import functools

import jax
import jax.numpy as jnp
import numpy as np
from jax import lax
from jax.experimental import pallas as pl
from jax.experimental.pallas import tpu as pltpu
from jax.experimental.pallas import tpu_sc as plsc

N = 10000
E = 320000
D_IN = 128
D_EDGE = 16
D_OUT = 128
KEEP_EDGE = 0.5

NPAD = 10240
NC, NS = 2, 16
NW = NC * NS
CHUNK = 80
ROWS_PT = NPAD // NS
ZROWS = 16

_mesh = plsc.VectorSubcoreMesh(core_axis_name="c", subcore_axis_name="s")

_KEEP_CACHE = {}


def _keep_ids():
    if "ids" not in _KEEP_CACHE:
        with jax.ensure_compile_time_eval():
            mask = np.asarray(
                jax.random.uniform(jax.random.key(12345), (E,)) < KEEP_EDGE)
        kept = np.where(mask)[0]
        dropped = np.where(~mask)[0]
        step = 2 * NW * CHUNK
        kp = ((len(kept) + step - 1) // step) * step
        pad = np.full(kp - len(kept), dropped[0], dtype=np.int64)
        _KEEP_CACHE["ids"] = np.concatenate([kept, pad]).astype(np.int32)
        _KEEP_CACHE["mask"] = mask
    return _KEEP_CACHE["ids"], _KEEP_CACHE["mask"]


def _sc_segment_sum(rows_hbm_spec, n_chunks, gather_table):
    scratch = [
        pltpu.VMEM((n_chunks, CHUNK), jnp.int32),
        pltpu.VMEM((CHUNK, D_IN), jnp.float32),
        pltpu.VMEM((ZROWS, D_IN), jnp.float32),
        pltpu.VMEM_SHARED((NPAD, D_IN), jnp.float32),
        pltpu.SemaphoreType.DMA,
    ]
    if gather_table:
        scratch.insert(0, pltpu.VMEM((n_chunks, CHUNK), jnp.int32))
        scratch.insert(2, pltpu.VMEM((CHUNK, D_IN), jnp.float32))

    @functools.partial(
        pl.kernel,
        out_type=jax.ShapeDtypeStruct((NC, NPAD, D_IN), jnp.float32),
        mesh=_mesh,
        scratch_types=scratch,
    )
    def k(*refs):
        if gather_table:
            (table_hbm, idx_hbm, dst_hbm, zeros_hbm, out_hbm,
             idx_v, dst_v, rows0_v, rows1_v, z_v, acc_sh, sem) = refs
        else:
            (vals_hbm, dst_hbm, zeros_hbm, out_hbm,
             dst_v, rows0_v, z_v, acc_sh, sem) = refs
        cid = lax.axis_index("c")
        sid = lax.axis_index("s")
        wid = sid * NC + cid

        pltpu.sync_copy(zeros_hbm, z_v)
        row0 = sid * ROWS_PT

        def zbody(j, carry):
            pltpu.sync_copy(z_v, acc_sh.at[pl.ds(row0 + j * ZROWS, ZROWS)])
            return carry

        lax.fori_loop(0, ROWS_PT // ZROWS, zbody, 0)
        plsc.subcore_barrier()

        pltpu.sync_copy(dst_hbm.at[wid], dst_v)

        if gather_table:
            pltpu.sync_copy(idx_hbm.at[wid], idx_v)
            bufs = (rows0_v, rows1_v)
            pltpu.async_copy(table_hbm.at[idx_v.at[0]], rows0_v, sem)

            def body(j, carry):
                for half in range(2):
                    i = 2 * j + half
                    buf, nbuf = bufs[half], bufs[1 - half]
                    pltpu.make_async_copy(
                        table_hbm.at[pl.ds(0, CHUNK)], buf, sem).wait()

                    @pl.when(i + 1 < n_chunks)
                    def _():
                        pltpu.async_copy(
                            table_hbm.at[idx_v.at[i + 1]], nbuf, sem)

                    pltpu.sync_copy(buf, acc_sh.at[dst_v.at[i]], add=True)
                return carry

            lax.fori_loop(0, n_chunks // 2, body, 0)
        else:

            def body(i, carry):
                pltpu.sync_copy(vals_hbm.at[wid * n_chunks + i], rows0_v)
                pltpu.sync_copy(rows0_v, acc_sh.at[dst_v.at[i]], add=True)
                return carry

            lax.fori_loop(0, n_chunks, body, 0)
        plsc.subcore_barrier()
        pltpu.sync_copy(acc_sh.at[pl.ds(row0, ROWS_PT)],
                        out_hbm.at[cid, pl.ds(row0, ROWS_PT)])

    return k


def _tc_matmul(a_parts, b_parts, wx, we_pad):
    BLK = 512

    def body(a_ref, b_ref, wx_ref, we_ref, o_ref):
        a = a_ref[0] + a_ref[1]
        b = b_ref[0] + b_ref[1]
        o_ref[...] = (
            jnp.dot(a, wx_ref[...], preferred_element_type=jnp.float32)
            + jnp.dot(b, we_ref[...], preferred_element_type=jnp.float32)
        )

    return pl.pallas_call(
        body,
        grid=(NPAD // BLK,),
        in_specs=[
            pl.BlockSpec((NC, BLK, D_IN), lambda i: (0, i, 0)),
            pl.BlockSpec((NC, BLK, D_IN), lambda i: (0, i, 0)),
            pl.BlockSpec((D_IN, D_OUT), lambda i: (0, 0)),
            pl.BlockSpec((D_IN, D_OUT), lambda i: (0, 0)),
        ],
        out_specs=pl.BlockSpec((BLK, D_OUT), lambda i: (i, 0)),
        out_shape=jax.ShapeDtypeStruct((NPAD, D_OUT), jnp.float32),
    )(a_parts, b_parts, wx, we_pad)


def kernel(x, edge_index, edge_attr, W):
    keep_np, mask_np = _keep_ids()
    keep = jnp.asarray(keep_np)
    n_chunks = len(keep_np) // (NW * CHUNK)
    mask_k = jnp.asarray(mask_np[keep_np])
    src_k = edge_index[0][keep].astype(jnp.int32).reshape(NW, n_chunks, CHUNK)
    dst_k = jnp.where(mask_k, edge_index[1][keep], N).astype(jnp.int32)
    dst_k = dst_k.reshape(NW, n_chunks, CHUNK)
    attr_k = jnp.pad(edge_attr[keep], ((0, 0), (0, D_IN - D_EDGE)))
    attr_k = attr_k.reshape(NW * n_chunks, CHUNK, D_IN)
    zeros = jnp.zeros((ZROWS, D_IN), jnp.float32)
    a_parts = _sc_segment_sum(None, n_chunks, True)(x, src_k, dst_k, zeros)
    b_parts = _sc_segment_sum(None, n_chunks, False)(attr_k, dst_k, zeros)
    wx = W[:, :D_IN].T
    we_pad = jnp.concatenate(
        [W[:, D_IN:].T, jnp.zeros((D_IN - D_EDGE, D_OUT), W.dtype)], axis=0)
    out = _tc_matmul(a_parts, b_parts, wx, we_pad)
    return out[:N]


# speedup vs baseline: 1.0359x; 1.0359x over previous
import functools

import jax
import jax.numpy as jnp
import numpy as np
from jax import lax
from jax.experimental import pallas as pl
from jax.experimental.pallas import tpu as pltpu
from jax.experimental.pallas import tpu_sc as plsc

N = 10000
E = 320000
D_IN = 128
D_EDGE = 16
D_OUT = 128
KEEP_EDGE = 0.5

NPAD = 10240
NC, NS = 2, 16
NW = NC * NS
CHUNK = 80
ROWS_PT = NPAD // NS
ZROWS = 16

_mesh = plsc.VectorSubcoreMesh(core_axis_name="c", subcore_axis_name="s")

_KEEP_CACHE = {}


def _keep_ids():
    if "ids" not in _KEEP_CACHE:
        with jax.ensure_compile_time_eval():
            mask = np.asarray(
                jax.random.uniform(jax.random.key(12345), (E,)) < KEEP_EDGE)
        kept = np.where(mask)[0]
        dropped = np.where(~mask)[0]
        step = NW * CHUNK
        kp = ((len(kept) + step - 1) // step) * step
        pad = np.full(kp - len(kept), dropped[0], dtype=np.int64)
        _KEEP_CACHE["ids"] = np.concatenate([kept, pad]).astype(np.int32)
        _KEEP_CACHE["mask"] = mask
    return _KEEP_CACHE["ids"], _KEEP_CACHE["mask"]


def _sc_segment_sum(rows_hbm_spec, n_chunks, gather_table):
    epw = n_chunks * CHUNK

    scratch = [
        pltpu.VMEM((CHUNK,), jnp.int32),
        pltpu.VMEM((CHUNK, D_IN), jnp.float32),
        pltpu.VMEM((ZROWS, D_IN), jnp.float32),
        pltpu.VMEM_SHARED((NPAD, D_IN), jnp.float32),
        pltpu.SemaphoreType.DMA,
    ]
    if gather_table:
        scratch.insert(0, pltpu.VMEM((CHUNK,), jnp.int32))

    @functools.partial(
        pl.kernel,
        out_type=jax.ShapeDtypeStruct((NC, NPAD, D_IN), jnp.float32),
        mesh=_mesh,
        scratch_types=scratch,
    )
    def k(*refs):
        if gather_table:
            (table_hbm, idx_hbm, dst_hbm, zeros_hbm, out_hbm,
             idx_v, dst_v, rows_v, z_v, acc_sh, sem) = refs
        else:
            (vals_hbm, dst_hbm, zeros_hbm, out_hbm,
             dst_v, rows_v, z_v, acc_sh, sem) = refs
        cid = lax.axis_index("c")
        sid = lax.axis_index("s")
        wid = sid * NC + cid

        pltpu.sync_copy(zeros_hbm, z_v)
        row0 = sid * ROWS_PT

        def zbody(j, carry):
            pltpu.sync_copy(z_v, acc_sh.at[pl.ds(row0 + j * ZROWS, ZROWS)])
            return carry

        lax.fori_loop(0, ROWS_PT // ZROWS, zbody, 0)
        plsc.subcore_barrier()

        def body(i, carry):
            base = wid * epw + i * CHUNK
            pltpu.sync_copy(dst_hbm.at[pl.ds(base, CHUNK)], dst_v)
            if gather_table:
                pltpu.sync_copy(idx_hbm.at[pl.ds(base, CHUNK)], idx_v)
                pltpu.async_copy(table_hbm.at[idx_v], rows_v, sem).wait()
            else:
                pltpu.sync_copy(vals_hbm.at[pl.ds(base, CHUNK)], rows_v)
            pltpu.sync_copy(rows_v, acc_sh.at[dst_v], add=True)
            return carry

        lax.fori_loop(0, n_chunks, body, 0)
        plsc.subcore_barrier()
        pltpu.sync_copy(acc_sh.at[pl.ds(row0, ROWS_PT)],
                        out_hbm.at[cid, pl.ds(row0, ROWS_PT)])

    return k


def _tc_matmul(a_parts, b_parts, wx, we_pad):
    BLK = 512

    def body(a_ref, b_ref, wx_ref, we_ref, o_ref):
        a = a_ref[0] + a_ref[1]
        b = b_ref[0] + b_ref[1]
        o_ref[...] = (
            jnp.dot(a, wx_ref[...], preferred_element_type=jnp.float32)
            + jnp.dot(b, we_ref[...], preferred_element_type=jnp.float32)
        )

    return pl.pallas_call(
        body,
        grid=(NPAD // BLK,),
        in_specs=[
            pl.BlockSpec((NC, BLK, D_IN), lambda i: (0, i, 0)),
            pl.BlockSpec((NC, BLK, D_IN), lambda i: (0, i, 0)),
            pl.BlockSpec((D_IN, D_OUT), lambda i: (0, 0)),
            pl.BlockSpec((D_IN, D_OUT), lambda i: (0, 0)),
        ],
        out_specs=pl.BlockSpec((BLK, D_OUT), lambda i: (i, 0)),
        out_shape=jax.ShapeDtypeStruct((NPAD, D_OUT), jnp.float32),
    )(a_parts, b_parts, wx, we_pad)


def kernel(x, edge_index, edge_attr, W):
    keep_np, mask_np = _keep_ids()
    keep = jnp.asarray(keep_np)
    n_chunks = len(keep_np) // (NW * CHUNK)
    mask_k = jnp.asarray(mask_np[keep_np])
    src_k = edge_index[0][keep].astype(jnp.int32)
    dst_k = jnp.where(mask_k, edge_index[1][keep], N).astype(jnp.int32)
    attr_k = jnp.pad(edge_attr[keep], ((0, 0), (0, D_IN - D_EDGE)))
    zeros = jnp.zeros((ZROWS, D_IN), jnp.float32)
    a_parts = _sc_segment_sum(None, n_chunks, True)(x, src_k, dst_k, zeros)
    b_parts = _sc_segment_sum(None, n_chunks, False)(attr_k, dst_k, zeros)
    wx = W[:, :D_IN].T
    we_pad = jnp.concatenate(
        [W[:, D_IN:].T, jnp.zeros((D_IN - D_EDGE, D_OUT), W.dtype)], axis=0)
    out = _tc_matmul(a_parts, b_parts, wx, we_pad)
    return out[:N]
